# baseline (speedup 1.0000x reference)
import functools

import jax
import jax.numpy as jnp
from jax import lax
from jax.experimental import pallas as pl
from jax.experimental.pallas import tpu as pltpu
from jax.experimental.pallas import tpu_sc as plsc

_W_ADJ = 0.4
_R_INV = 0.1

_VPAD = 100352
_NCORES = 2
_NW = 16 * _NCORES
_CH = 32


def _make_sc_kernel(batch, seq):
    spw = batch // _NW
    nchunk = spw // _CH
    mesh = plsc.VectorSubcoreMesh(core_axis_name="c", subcore_axis_name="s",
                                  num_cores=_NCORES)
    ngrp = seq // 16
    rem = seq - ngrp * 16

    vs = _VPAD // 16
    sub = vs // 4

    @functools.partial(
        pl.kernel,
        mesh=mesh,
        out_type=jax.ShapeDtypeStruct((batch * 2,), jnp.float32),
        scratch_types=[
            pltpu.VMEM((_VPAD,), jnp.int32),
            pltpu.VMEM((_CH, seq), jnp.int32),
            pltpu.VMEM((_CH, seq), jnp.int32),
            pltpu.VMEM((spw * 2 + 16,), jnp.float32),
            pltpu.VMEM((sub,), jnp.float32),
            pltpu.VMEM((2 * sub,), jnp.float32),
            pltpu.VMEM_SHARED((_VPAD,), jnp.int32),
            pltpu.SemaphoreType.DMA,
            pltpu.SemaphoreType.DMA,
        ],
        compiler_params=pltpu.CompilerParams(needs_layout_passes=False),
    )
    def sc_kernel(w_hbm, r_hbm, fi_hbm, out_hbm, table_v, idx_a, idx_b,
                  stage_v, wbuf, rbuf, table_sh, sem_a, sem_b):
        cid = lax.axis_index("c")
        sid = lax.axis_index("s")
        wid = sid * _NCORES + cid
        bufs = (idx_a, idx_b)
        sems = (sem_a, sem_b)

        def issue(g):
            row = wid * spw + g * _CH
            return pltpu.async_copy(
                fi_hbm.at[pl.ds(row, _CH), :],
                bufs[g % 2],
                sems[g % 2])

        lanes = lax.iota(jnp.int32, 16)
        mrem = lanes >= (16 - rem)
        m2 = lanes < 2
        zero = jnp.zeros((16,), jnp.float32)
        hi_mask = jnp.int32(-65536)

        pending = issue(0)

        for s_i in range(vs // sub):
            row0 = sid * vs + s_i * sub
            pltpu.sync_copy(w_hbm.at[pl.ds(row0, sub)], wbuf)
            pltpu.sync_copy(r_hbm.at[pl.ds(2 * row0, 2 * sub)], rbuf)

            @plsc.parallel_loop(0, sub // 16, unroll=2)
            def pack(k, s_i=s_i):
                wv = wbuf[pl.ds(k * 16, 16)] + jnp.float32(_W_ADJ)
                ri = 32 * k + 2 * lanes
                r0 = plsc.load_gather(rbuf, [ri])
                r1 = plsc.load_gather(rbuf, [ri + 1])
                p0 = (wv * r0) * jnp.float32(_R_INV)
                p1 = (wv * r1) * jnp.float32(_R_INV)
                b0 = lax.bitcast_convert_type(p0, jnp.int32)
                b1 = lax.bitcast_convert_type(p1, jnp.int32)
                h0 = lax.shift_right_logical(
                    b0 + 0x7FFF + ((b0 >> 16) & 1), 16)
                h1 = lax.shift_right_logical(
                    b1 + 0x7FFF + ((b1 >> 16) & 1), 16)
                table_v[pl.ds(sid * vs + s_i * sub + k * 16, 16)] = (
                    h0 | (h1 << 16))

        pltpu.sync_copy(table_v.at[pl.ds(sid * vs, vs)],
                        table_sh.at[pl.ds(sid * vs, vs)])
        plsc.subcore_barrier()
        pltpu.sync_copy(table_sh, table_v)

        for g in range(nchunk):
            idx_v = bufs[g % 2]
            pending.wait()
            if g + 1 < nchunk:
                pending = issue(g + 1)

            @plsc.parallel_loop(0, _CH, unroll=2)
            def body(i, g=g, idx_v=idx_v):
                acc0a = acc0b = acc1a = acc1b = zero
                for j in range(ngrp + (1 if rem else 0)):
                    if j == ngrp:
                        iv = idx_v[i, pl.ds(seq - 16, 16)]
                        iv = jnp.where(mrem, iv, 0)
                    else:
                        iv = idx_v[i, pl.ds(16 * j, 16)]
                    word = plsc.load_gather(table_v, [iv])
                    p0 = lax.bitcast_convert_type(word << 16, jnp.float32)
                    p1 = lax.bitcast_convert_type(word & hi_mask, jnp.float32)
                    if j % 2 == 0:
                        acc0a = acc0a + p0
                        acc1a = acc1a + p1
                    else:
                        acc0b = acc0b + p0
                        acc1b = acc1b + p1
                t0 = jnp.sum(acc0a + acc0b)
                t1 = jnp.sum(acc1a + acc1b)
                v = jnp.where(lanes == 0, t0, t1)
                pos = (g * _CH + i) * 2 + lanes
                plsc.store_scatter(stage_v, [pos], v, mask=m2)

        pltpu.sync_copy(stage_v.at[pl.ds(0, spw * 2)],
                        out_hbm.at[pl.ds(wid * spw * 2, spw * 2)])

    return sc_kernel


def kernel(W, R, feat_idx):
    batch, seq = feat_idx.shape
    pad = _VPAD - W.shape[0]
    wflat = jnp.pad(W[:, 0], (0, pad))
    rflat = jnp.pad(R, ((0, pad), (0, 0))).reshape(2 * _VPAD)
    out = _make_sc_kernel(batch, seq)(wflat, rflat, feat_idx)
    return out.reshape(batch, 2)


# speedup vs baseline: 1.9549x; 1.9549x over previous
import functools

import jax
import jax.numpy as jnp
from jax import lax
from jax.experimental import pallas as pl
from jax.experimental.pallas import tpu as pltpu
from jax.experimental.pallas import tpu_sc as plsc

_W_ADJ = 0.4
_R_INV = 0.1

_VPAD = 100352
_NCORES = 2
_NW = 16 * _NCORES
_CH = 32


def _make_sc_kernel(batch, seq):
    spw = batch // _NW
    nchunk = spw // _CH
    mesh = plsc.VectorSubcoreMesh(core_axis_name="c", subcore_axis_name="s",
                                  num_cores=_NCORES)
    ngrp = seq // 16
    rem = seq - ngrp * 16

    vs = _VPAD // 16
    sub = vs // 4

    @functools.partial(
        pl.kernel,
        mesh=mesh,
        out_type=jax.ShapeDtypeStruct((batch * 2,), jnp.float32),
        scratch_types=[
            pltpu.VMEM((_VPAD,), jnp.int32),
            pltpu.VMEM((_CH, seq), jnp.int32),
            pltpu.VMEM((_CH, seq), jnp.int32),
            pltpu.VMEM((spw * 2 + 16,), jnp.float32),
            pltpu.VMEM((sub,), jnp.float32),
            pltpu.VMEM((sub,), jnp.float32),
            pltpu.VMEM((sub,), jnp.float32),
            pltpu.VMEM_SHARED((_VPAD,), jnp.int32),
            pltpu.SemaphoreType.DMA,
            pltpu.SemaphoreType.DMA,
        ],
        compiler_params=pltpu.CompilerParams(needs_layout_passes=False),
    )
    def sc_kernel(w_hbm, r0_hbm, r1_hbm, fi_hbm, out_hbm, table_v, idx_a,
                  idx_b, stage_v, wbuf, r0buf, r1buf, table_sh, sem_a, sem_b):
        cid = lax.axis_index("c")
        sid = lax.axis_index("s")
        wid = sid * _NCORES + cid
        bufs = (idx_a, idx_b)
        sems = (sem_a, sem_b)

        def issue(g):
            row = wid * spw + g * _CH
            return pltpu.async_copy(
                fi_hbm.at[pl.ds(row, _CH), :],
                bufs[g % 2],
                sems[g % 2])

        lanes = lax.iota(jnp.int32, 16)
        mrem = lanes >= (16 - rem)
        m2 = lanes < 2
        zero = jnp.zeros((16,), jnp.float32)
        hi_mask = jnp.int32(-65536)

        pending = issue(0)

        for s_i in range(vs // sub):
            row0 = sid * vs + s_i * sub
            pltpu.sync_copy(w_hbm.at[pl.ds(row0, sub)], wbuf)
            pltpu.sync_copy(r0_hbm.at[pl.ds(row0, sub)], r0buf)
            pltpu.sync_copy(r1_hbm.at[pl.ds(row0, sub)], r1buf)

            @plsc.parallel_loop(0, sub // 16, unroll=2)
            def pack(k, s_i=s_i):
                wv = wbuf[pl.ds(k * 16, 16)] + jnp.float32(_W_ADJ)
                r0 = r0buf[pl.ds(k * 16, 16)]
                r1 = r1buf[pl.ds(k * 16, 16)]
                p0 = (wv * r0) * jnp.float32(_R_INV)
                p1 = (wv * r1) * jnp.float32(_R_INV)
                b0 = lax.bitcast_convert_type(p0, jnp.int32)
                b1 = lax.bitcast_convert_type(p1, jnp.int32)
                h0 = lax.shift_right_logical(
                    b0 + 0x7FFF + ((b0 >> 16) & 1), 16)
                h1 = lax.shift_right_logical(
                    b1 + 0x7FFF + ((b1 >> 16) & 1), 16)
                table_v[pl.ds(sid * vs + s_i * sub + k * 16, 16)] = (
                    h0 | (h1 << 16))

        pltpu.sync_copy(table_v.at[pl.ds(sid * vs, vs)],
                        table_sh.at[pl.ds(sid * vs, vs)])
        plsc.subcore_barrier()
        pltpu.sync_copy(table_sh, table_v)

        for g in range(nchunk):
            idx_v = bufs[g % 2]
            pending.wait()
            if g + 1 < nchunk:
                pending = issue(g + 1)

            @plsc.parallel_loop(0, _CH, unroll=2)
            def body(i, g=g, idx_v=idx_v):
                acc0a = acc0b = acc1a = acc1b = zero
                for j in range(ngrp + (1 if rem else 0)):
                    if j == ngrp:
                        iv = idx_v[i, pl.ds(seq - 16, 16)]
                        iv = jnp.where(mrem, iv, 0)
                    else:
                        iv = idx_v[i, pl.ds(16 * j, 16)]
                    word = plsc.load_gather(table_v, [iv])
                    p0 = lax.bitcast_convert_type(word << 16, jnp.float32)
                    p1 = lax.bitcast_convert_type(word & hi_mask, jnp.float32)
                    if j % 2 == 0:
                        acc0a = acc0a + p0
                        acc1a = acc1a + p1
                    else:
                        acc0b = acc0b + p0
                        acc1b = acc1b + p1
                t0 = jnp.sum(acc0a + acc0b)
                t1 = jnp.sum(acc1a + acc1b)
                v = jnp.where(lanes == 0, t0, t1)
                pos = (g * _CH + i) * 2 + lanes
                plsc.store_scatter(stage_v, [pos], v, mask=m2)

        pltpu.sync_copy(stage_v.at[pl.ds(0, spw * 2)],
                        out_hbm.at[pl.ds(wid * spw * 2, spw * 2)])

    return sc_kernel


def kernel(W, R, feat_idx):
    batch, seq = feat_idx.shape
    pad = _VPAD - W.shape[0]
    wflat = jnp.pad(W[:, 0], (0, pad))
    r0 = jnp.pad(R[:, 0], (0, pad))
    r1 = jnp.pad(R[:, 1], (0, pad))
    out = _make_sc_kernel(batch, seq)(wflat, r0, r1, feat_idx)
    return out.reshape(batch, 2)


# speedup vs baseline: 2.1162x; 1.0825x over previous
import functools

import jax
import jax.numpy as jnp
from jax import lax
from jax.experimental import pallas as pl
from jax.experimental.pallas import tpu as pltpu
from jax.experimental.pallas import tpu_sc as plsc

_W_ADJ = 0.4
_R_INV = 0.1

_VPAD = 100352
_NW = 32
_CH = 32


def _pack_body(w_ref, r0_ref, r1_ref, o_ref):
    w = w_ref[...] + jnp.float32(_W_ADJ)
    p0 = (w * r0_ref[...]) * jnp.float32(_R_INV)
    p1 = (w * r1_ref[...]) * jnp.float32(_R_INV)
    b0 = lax.bitcast_convert_type(p0.astype(jnp.bfloat16), jnp.uint16).astype(jnp.uint32)
    b1 = lax.bitcast_convert_type(p1.astype(jnp.bfloat16), jnp.uint16).astype(jnp.uint32)
    o_ref[...] = lax.bitcast_convert_type(b0 | (b1 << jnp.uint32(16)), jnp.int32)


def _pack_table(wcol, r0, r1):
    return pl.pallas_call(
        _pack_body,
        out_shape=jax.ShapeDtypeStruct(wcol.shape, jnp.int32),
    )(wcol, r0, r1)


def _make_sc_kernel(batch, seq):
    spw = batch // _NW
    nchunk = spw // _CH
    mesh = plsc.VectorSubcoreMesh(core_axis_name="c", subcore_axis_name="s")
    ngrp = seq // 16
    rem = seq - ngrp * 16

    @functools.partial(
        pl.kernel,
        mesh=mesh,
        out_type=jax.ShapeDtypeStruct((batch * 2,), jnp.float32),
        scratch_types=[
            pltpu.VMEM((_VPAD,), jnp.int32),
            pltpu.VMEM((_CH, seq), jnp.int32),
            pltpu.VMEM((_CH, seq), jnp.int32),
            pltpu.VMEM((spw * 2 + 16,), jnp.float32),
            pltpu.VMEM_SHARED((_VPAD,), jnp.int32),
            pltpu.SemaphoreType.DMA,
            pltpu.SemaphoreType.DMA,
        ],
        compiler_params=pltpu.CompilerParams(needs_layout_passes=False),
    )
    def sc_kernel(packed_hbm, fi_hbm, out_hbm, table_v, idx_a, idx_b,
                  stage_v, table_sh, sem_a, sem_b):
        cid = lax.axis_index("c")
        sid = lax.axis_index("s")
        wid = sid * 2 + cid
        bufs = (idx_a, idx_b)
        sems = (sem_a, sem_b)

        def issue(g):
            row = wid * spw + g * _CH
            return pltpu.async_copy(
                fi_hbm.at[pl.ds(row, _CH), :],
                bufs[g % 2],
                sems[g % 2])

        lanes = lax.iota(jnp.int32, 16)
        mrem = lanes >= (16 - rem)
        m2 = lanes < 2
        zero = jnp.zeros((16,), jnp.float32)
        hi_mask = jnp.int32(-65536)

        pending = issue(0)

        @pl.when(sid == 0)
        def _():
            pltpu.sync_copy(packed_hbm, table_sh)

        plsc.subcore_barrier()
        pltpu.sync_copy(table_sh, table_v)

        for g in range(nchunk):
            idx_v = bufs[g % 2]
            pending.wait()
            if g + 1 < nchunk:
                pending = issue(g + 1)

            @plsc.parallel_loop(0, _CH, unroll=2)
            def body(i, g=g, idx_v=idx_v):
                acc0a = acc0b = acc1a = acc1b = zero
                for j in range(ngrp + (1 if rem else 0)):
                    if j == ngrp:
                        iv = idx_v[i, pl.ds(seq - 16, 16)]
                        iv = jnp.where(mrem, iv, 0)
                    else:
                        iv = idx_v[i, pl.ds(16 * j, 16)]
                    word = plsc.load_gather(table_v, [iv])
                    p0 = lax.bitcast_convert_type(word << 16, jnp.float32)
                    p1 = lax.bitcast_convert_type(word & hi_mask, jnp.float32)
                    if j % 2 == 0:
                        acc0a = acc0a + p0
                        acc1a = acc1a + p1
                    else:
                        acc0b = acc0b + p0
                        acc1b = acc1b + p1
                t0 = jnp.sum(acc0a + acc0b)
                t1 = jnp.sum(acc1a + acc1b)
                v = jnp.where(lanes == 0, t0, t1)
                pos = (g * _CH + i) * 2 + lanes
                plsc.store_scatter(stage_v, [pos], v, mask=m2)

        pltpu.sync_copy(stage_v.at[pl.ds(0, spw * 2)],
                        out_hbm.at[pl.ds(wid * spw * 2, spw * 2)])

    return sc_kernel


def kernel(W, R, feat_idx):
    batch, seq = feat_idx.shape
    pad = _VPAD - W.shape[0]
    wcol = jnp.pad(W[:, 0], (0, pad)).reshape(-1, 128)
    r0 = jnp.pad(R[:, 0], (0, pad)).reshape(-1, 128)
    r1 = jnp.pad(R[:, 1], (0, pad)).reshape(-1, 128)
    packed = _pack_table(wcol, r0, r1).reshape(_VPAD)
    out = _make_sc_kernel(batch, seq)(packed, feat_idx)
    return out.reshape(batch, 2)


# speedup vs baseline: 2.1176x; 1.0006x over previous
import functools

import jax
import jax.numpy as jnp
from jax import lax
from jax.experimental import pallas as pl
from jax.experimental.pallas import tpu as pltpu
from jax.experimental.pallas import tpu_sc as plsc

_W_ADJ = 0.4
_R_INV = 0.1

_VPAD = 100352
_NW = 32
_CH = 32


def _pack_body(w_ref, r0_ref, r1_ref, o_ref):
    w = w_ref[...] + jnp.float32(_W_ADJ)
    p0 = (w * r0_ref[...]) * jnp.float32(_R_INV)
    p1 = (w * r1_ref[...]) * jnp.float32(_R_INV)
    b0 = lax.bitcast_convert_type(p0.astype(jnp.bfloat16), jnp.uint16).astype(jnp.uint32)
    b1 = lax.bitcast_convert_type(p1.astype(jnp.bfloat16), jnp.uint16).astype(jnp.uint32)
    o_ref[...] = lax.bitcast_convert_type(b0 | (b1 << jnp.uint32(16)), jnp.int32)


def _pack_table(wcol, r0, r1):
    return pl.pallas_call(
        _pack_body,
        out_shape=jax.ShapeDtypeStruct(wcol.shape, jnp.int32),
    )(wcol, r0, r1)


def _make_sc_kernel(batch, seq):
    spw = batch // _NW
    nchunk = spw // _CH
    mesh = plsc.VectorSubcoreMesh(core_axis_name="c", subcore_axis_name="s")
    ngrp = seq // 16
    rem = seq - ngrp * 16

    @functools.partial(
        pl.kernel,
        mesh=mesh,
        out_type=jax.ShapeDtypeStruct((batch * 2,), jnp.float32),
        scratch_types=[
            pltpu.VMEM((_VPAD // 128, 128), jnp.int32),
            pltpu.VMEM((_CH, seq), jnp.int32),
            pltpu.VMEM((_CH, seq), jnp.int32),
            pltpu.VMEM((spw * 2 + 16,), jnp.float32),
            pltpu.VMEM_SHARED((_VPAD // 128, 128), jnp.int32),
            pltpu.SemaphoreType.DMA,
            pltpu.SemaphoreType.DMA,
        ],
        compiler_params=pltpu.CompilerParams(needs_layout_passes=False),
    )
    def sc_kernel(packed_hbm, fi_hbm, out_hbm, table_v, idx_a, idx_b,
                  stage_v, table_sh, sem_a, sem_b):
        cid = lax.axis_index("c")
        sid = lax.axis_index("s")
        wid = sid * 2 + cid
        bufs = (idx_a, idx_b)
        sems = (sem_a, sem_b)

        def issue(g):
            row = wid * spw + g * _CH
            return pltpu.async_copy(
                fi_hbm.at[pl.ds(row, _CH), :],
                bufs[g % 2],
                sems[g % 2])

        lanes = lax.iota(jnp.int32, 16)
        mrem = lanes >= (16 - rem)
        m2 = lanes < 2
        zero = jnp.zeros((16,), jnp.float32)
        hi_mask = jnp.int32(-65536)

        pending = issue(0)

        @pl.when(sid == 0)
        def _():
            pltpu.sync_copy(packed_hbm, table_sh)

        plsc.subcore_barrier()
        pltpu.sync_copy(table_sh, table_v)

        for g in range(nchunk):
            idx_v = bufs[g % 2]
            pending.wait()
            if g + 1 < nchunk:
                pending = issue(g + 1)

            @plsc.parallel_loop(0, _CH, unroll=2)
            def body(i, g=g, idx_v=idx_v):
                acc0a = acc0b = acc1a = acc1b = zero
                for j in range(ngrp + (1 if rem else 0)):
                    if j == ngrp:
                        iv = idx_v[i, pl.ds(seq - 16, 16)]
                        iv = jnp.where(mrem, iv, 0)
                    else:
                        iv = idx_v[i, pl.ds(16 * j, 16)]
                    word = plsc.load_gather(table_v, [iv >> 7, iv & 127])
                    p0 = lax.bitcast_convert_type(word << 16, jnp.float32)
                    p1 = lax.bitcast_convert_type(word & hi_mask, jnp.float32)
                    if j % 2 == 0:
                        acc0a = acc0a + p0
                        acc1a = acc1a + p1
                    else:
                        acc0b = acc0b + p0
                        acc1b = acc1b + p1
                t0 = jnp.sum(acc0a + acc0b)
                t1 = jnp.sum(acc1a + acc1b)
                v = jnp.where(lanes == 0, t0, t1)
                pos = (g * _CH + i) * 2 + lanes
                plsc.store_scatter(stage_v, [pos], v, mask=m2)

        pltpu.sync_copy(stage_v.at[pl.ds(0, spw * 2)],
                        out_hbm.at[pl.ds(wid * spw * 2, spw * 2)])

    return sc_kernel


def kernel(W, R, feat_idx):
    batch, seq = feat_idx.shape
    pad = _VPAD - W.shape[0]
    wcol = jnp.pad(W[:, 0], (0, pad)).reshape(-1, 128)
    r0 = jnp.pad(R[:, 0], (0, pad)).reshape(-1, 128)
    r1 = jnp.pad(R[:, 1], (0, pad)).reshape(-1, 128)
    packed = _pack_table(wcol, r0, r1)
    out = _make_sc_kernel(batch, seq)(packed, feat_idx)
    return out.reshape(batch, 2)
